# baseline (speedup 1.0000x reference)
import jax
import jax.numpy as jnp
from jax.experimental import pallas as pl

G = 30
C = 10
BIG = 1000000000.0


def _body(g3_ref, m3_ref, hl_ref, wl_ref, cell_ref):
    g = g3_ref[...]
    m = m3_ref[...]
    safe = jnp.where(m, g, 0)
    bb = g.shape[0]
    iota_c = jax.lax.broadcasted_iota(jnp.int32, (bb, G, G, C), 3)
    eq = safe[..., None] == iota_c
    cell_ref[...] = jnp.where(eq, BIG, -BIG)

    row_any = jnp.any(m, axis=2)
    col_any = jnp.any(m, axis=1)
    h = jnp.sum(row_any.astype(jnp.int32), axis=1) - 1
    w = jnp.sum(col_any.astype(jnp.int32), axis=1) - 1
    h = jnp.where(h < 0, h + G, h)
    w = jnp.where(w < 0, w + G, w)
    iot = jax.lax.broadcasted_iota(jnp.int32, (bb, G), 1)
    hl_ref[...] = jnp.where(iot == h[:, None], BIG, -BIG)
    wl_ref[...] = jnp.where(iot == w[:, None], BIG, -BIG)


def _build(B, Bb, interpret=False):
    grid = (B // Bb,)
    return pl.pallas_call(
        _body,
        grid=grid,
        in_specs=[
            pl.BlockSpec((Bb, G, G), lambda i: (i, 0, 0)),
            pl.BlockSpec((Bb, G, G), lambda i: (i, 0, 0)),
        ],
        out_specs=[
            pl.BlockSpec((Bb, G), lambda i: (i, 0)),
            pl.BlockSpec((Bb, G), lambda i: (i, 0)),
            pl.BlockSpec((Bb, G, G, C), lambda i: (i, 0, 0, 0)),
        ],
        out_shape=[
            jax.ShapeDtypeStruct((B, G), jnp.float32),
            jax.ShapeDtypeStruct((B, G), jnp.float32),
            jax.ShapeDtypeStruct((B, G, G, C), jnp.float32),
        ],
        interpret=interpret,
    )


def kernel(demo_input_grids, demo_input_masks, demo_output_grids,
           demo_output_masks, demo_mask, query_input_grid, query_input_mask):
    del demo_input_grids, demo_input_masks, demo_output_grids
    del demo_output_masks, demo_mask
    B = query_input_grid.shape[0]
    Bb = 32
    hl, wl, cell = _build(B, Bb)(query_input_grid, query_input_mask)
    return (hl, wl, cell)


# speedup vs baseline: 26.7861x; 26.7861x over previous
import jax
import jax.numpy as jnp
from jax.experimental import pallas as pl

G = 30
C = 10
BIG = 1000000000.0


def _body(gt_ref, mt_ref, hlt_ref, wlt_ref, cellt_ref):
    g = gt_ref[...]
    m = mt_ref[...]
    bb = g.shape[-1]
    safe = jnp.where(m, g, 0)
    iota_c = jax.lax.broadcasted_iota(jnp.int32, (G, C, G, bb), 1)
    eq = safe[:, None, :, :] == iota_c
    cellt_ref[...] = jnp.where(eq, BIG, -BIG)

    row_any = jnp.any(m, axis=1)
    col_any = jnp.any(m, axis=0)
    h = jnp.sum(row_any.astype(jnp.int32), axis=0) - 1
    w = jnp.sum(col_any.astype(jnp.int32), axis=0) - 1
    h = jnp.where(h < 0, h + G, h)
    w = jnp.where(w < 0, w + G, w)
    iot = jax.lax.broadcasted_iota(jnp.int32, (G, bb), 0)
    hlt_ref[...] = jnp.where(iot == h[None, :], BIG, -BIG)
    wlt_ref[...] = jnp.where(iot == w[None, :], BIG, -BIG)


def _build(B, Bb, interpret=False):
    grid = (B // Bb,)
    return pl.pallas_call(
        _body,
        grid=grid,
        in_specs=[
            pl.BlockSpec((G, G, Bb), lambda i: (0, 0, i)),
            pl.BlockSpec((G, G, Bb), lambda i: (0, 0, i)),
        ],
        out_specs=[
            pl.BlockSpec((G, Bb), lambda i: (0, i)),
            pl.BlockSpec((G, Bb), lambda i: (0, i)),
            pl.BlockSpec((G, C, G, Bb), lambda i: (0, 0, 0, i)),
        ],
        out_shape=[
            jax.ShapeDtypeStruct((G, B), jnp.float32),
            jax.ShapeDtypeStruct((G, B), jnp.float32),
            jax.ShapeDtypeStruct((G, C, G, B), jnp.float32),
        ],
        interpret=interpret,
    )


def kernel(demo_input_grids, demo_input_masks, demo_output_grids,
           demo_output_masks, demo_mask, query_input_grid, query_input_mask):
    del demo_input_grids, demo_input_masks, demo_output_grids
    del demo_output_masks, demo_mask
    B = query_input_grid.shape[0]
    Bb = 256
    gt = jnp.transpose(query_input_grid, (1, 2, 0))
    mt = jnp.transpose(query_input_mask, (1, 2, 0))
    hlt, wlt, cellt = _build(B, Bb)(gt, mt)
    return (hlt.T, wlt.T, jnp.transpose(cellt, (3, 0, 2, 1)))


# speedup vs baseline: 27.0730x; 1.0107x over previous
import jax
import jax.numpy as jnp
from jax.experimental import pallas as pl

G = 30
C = 10
BIG = 1000000000.0


def _body(gt_ref, mt_ref, hlt_ref, wlt_ref, cellt_ref):
    g = gt_ref[...]
    m = mt_ref[...]
    bb = g.shape[-1]
    safe = jnp.where(m, g, 0)
    iota_c = jax.lax.broadcasted_iota(jnp.int32, (G, C, G, bb), 1)
    eq = safe[:, None, :, :] == iota_c
    cellt_ref[...] = jnp.where(eq, BIG, -BIG)

    row_any = jnp.any(m, axis=1)
    col_any = jnp.any(m, axis=0)
    h = jnp.sum(row_any.astype(jnp.int32), axis=0) - 1
    w = jnp.sum(col_any.astype(jnp.int32), axis=0) - 1
    h = jnp.where(h < 0, h + G, h)
    w = jnp.where(w < 0, w + G, w)
    iot = jax.lax.broadcasted_iota(jnp.int32, (G, bb), 0)
    hlt_ref[...] = jnp.where(iot == h[None, :], BIG, -BIG)
    wlt_ref[...] = jnp.where(iot == w[None, :], BIG, -BIG)


def _build(B, Bb, interpret=False):
    grid = (B // Bb,)
    return pl.pallas_call(
        _body,
        grid=grid,
        in_specs=[
            pl.BlockSpec((G, G, Bb), lambda i: (0, 0, i)),
            pl.BlockSpec((G, G, Bb), lambda i: (0, 0, i)),
        ],
        out_specs=[
            pl.BlockSpec((G, Bb), lambda i: (0, i)),
            pl.BlockSpec((G, Bb), lambda i: (0, i)),
            pl.BlockSpec((G, C, G, Bb), lambda i: (0, 0, 0, i)),
        ],
        out_shape=[
            jax.ShapeDtypeStruct((G, B), jnp.float32),
            jax.ShapeDtypeStruct((G, B), jnp.float32),
            jax.ShapeDtypeStruct((G, C, G, B), jnp.float32),
        ],
        interpret=interpret,
    )


def kernel(demo_input_grids, demo_input_masks, demo_output_grids,
           demo_output_masks, demo_mask, query_input_grid, query_input_mask):
    del demo_input_grids, demo_input_masks, demo_output_grids
    del demo_output_masks, demo_mask
    B = query_input_grid.shape[0]
    Bb = 512
    gt = jnp.transpose(query_input_grid, (1, 2, 0))
    mt = jnp.transpose(query_input_mask, (1, 2, 0))
    hlt, wlt, cellt = _build(B, Bb)(gt, mt)
    return (hlt.T, wlt.T, jnp.transpose(cellt, (3, 0, 2, 1)))
